# baseline (speedup 1.0000x reference)
import functools

import jax
import jax.numpy as jnp
import numpy as np
from jax import lax
from jax.experimental import pallas as pl
from jax.experimental.pallas import tpu as pltpu
from jax.experimental.pallas import tpu_sc as plsc

_NUM_LEVELS = 16
_HASHMAP_SIZE = 2 ** 19
_N = 524288
_PRIME_X, _PRIME_Y, _PRIME_Z = 73856093, 19349663, 83492791

_RES, _OFF, _SIZE = [], [], []
_t = 0
for _l in range(_NUM_LEVELS):
    _r = min(int(16 * (2.0 ** _l)), 512)
    _RES.append(_r)
    _OFF.append(_t)
    _SIZE.append(min(_r ** 3, _HASHMAP_SIZE))
    _t += _SIZE[-1]
_TOTAL = _t

_NC, _NS = 2, 16
_NW = _NC * _NS
_G = 16
_PPW = _N // _NW
_NGROUPS = _PPW // _G
_BLK = 4

_GEOM_LEVELS = [[0], [1], [2], [3], [4], list(range(5, _NUM_LEVELS))]


def _encode_body(positions, emb_blocks, params, out,
                 pos_v, par_v, idx_v, w_v, col_v, rows_v, out_v, sem):
    wid = lax.axis_index("s") * np.int32(_NC) + lax.axis_index("c")
    wbase = wid * np.int32(_PPW)

    pltpu.sync_copy(params, par_v)
    iota = lax.iota(jnp.int32, 16)
    amin = [par_v[i, :] for i in range(3)]
    ainv = [par_v[3 + i, :] for i in range(3)]
    rowids = [iota + np.int32(c * 16) for c in range(8)]
    one_i = jnp.full((16,), 1, jnp.int32)

    def group(g, base):
        base = pl.multiple_of(base, _G)
        pltpu.sync_copy(positions.at[pl.ds(base, _G)], pos_v)

        u = []
        for ax in range(3):
            p = plsc.load_gather(pos_v, [iota, jnp.full((16,), ax, jnp.int32)])
            u.append(jnp.clip((p - amin[ax]) * ainv[ax], 0.0, 1.0))

        for gi, levels in enumerate(_GEOM_LEVELS):
            res = _RES[levels[0]]
            mask = _SIZE[levels[0]] - 1
            s = [u[ax] * jnp.float32(res - 1) for ax in range(3)]
            c0 = [sv.astype(jnp.int32) for sv in s]
            f = [s[ax] - c0[ax].astype(jnp.float32) for ax in range(3)]
            c1 = [jnp.minimum(c0[ax] + np.int32(1), np.int32(res - 1))
                  for ax in range(3)]
            mult = (_PRIME_X, _PRIME_Y, _PRIME_Z)
            h0 = [c0[ax] * np.int32(mult[ax]) for ax in range(3)]
            h1 = [c1[ax] * np.int32(mult[ax]) for ax in range(3)]
            w0 = [1.0 - f[ax] for ax in range(3)]
            for dx in range(2):
                hx = h1[0] if dx else h0[0]
                wx = f[0] if dx else w0[0]
                for dy in range(2):
                    hxy = hx + (h1[1] if dy else h0[1])
                    wxy = wx * (f[1] if dy else w0[1])
                    for dz in range(2):
                        cidx = dx * 4 + dy * 2 + dz
                        hm = (hxy + (h1[2] if dz else h0[2])) & np.int32(mask)
                        w = wxy * (f[2] if dz else w0[2])
                        blk = lax.shift_right_logical(hm, np.int32(2))
                        col = lax.shift_left(hm & np.int32(_BLK - 1), np.int32(1))
                        for lvl in levels:
                            idx_v[lvl, cidx * 16:(cidx + 1) * 16] = (
                                blk + np.int32(_OFF[lvl] // _BLK))
                        w_v[gi, cidx * 16:(cidx + 1) * 16] = w
                        col_v[gi, cidx * 16:(cidx + 1) * 16] = col

        copies = [pltpu.async_copy(emb_blocks.at[idx_v.at[np.int32(l)]],
                                   rows_v.at[np.int32(l)], sem)
                  for l in range(_NUM_LEVELS)]
        for cp in copies:
            cp.wait()

        for gi, levels in enumerate(_GEOM_LEVELS):
            acc0 = [jnp.zeros((16,), jnp.float32) for _ in levels]
            acc1 = [jnp.zeros((16,), jnp.float32) for _ in levels]
            for cidx in range(8):
                w = w_v[gi, cidx * 16:(cidx + 1) * 16]
                col0 = col_v[gi, cidx * 16:(cidx + 1) * 16]
                col1 = col0 + one_i
                for li, lvl in enumerate(levels):
                    lsp = jnp.full((16,), lvl, jnp.int32)
                    f0 = plsc.load_gather(rows_v, [lsp, rowids[cidx], col0])
                    f1 = plsc.load_gather(rows_v, [lsp, rowids[cidx], col1])
                    acc0[li] = acc0[li] + w * f0
                    acc1[li] = acc1[li] + w * f1
            for li, lvl in enumerate(levels):
                plsc.store_scatter(out_v, [iota, jnp.full((16,), 2 * lvl, jnp.int32)],
                                   acc0[li])
                plsc.store_scatter(out_v, [iota, jnp.full((16,), 2 * lvl + 1, jnp.int32)],
                                   acc1[li])

        pltpu.sync_copy(out_v, out.at[pl.ds(base, _G)])
        return base + np.int32(_G)

    lax.fori_loop(0, _NGROUPS, group, wbase)


@functools.lru_cache(maxsize=1)
def _build():
    mesh = plsc.VectorSubcoreMesh(core_axis_name="c", subcore_axis_name="s")
    return functools.partial(
        pl.kernel,
        out_type=jax.ShapeDtypeStruct((_N, 2 * _NUM_LEVELS), jnp.float32),
        mesh=mesh,
        compiler_params=pltpu.CompilerParams(needs_layout_passes=False,
                                             use_tc_tiling_on_sc=False),
        scratch_types=[
            pltpu.VMEM((_G, 3), jnp.float32),
            pltpu.VMEM((6, 16), jnp.float32),
            pltpu.VMEM((_NUM_LEVELS, 128), jnp.int32),
            pltpu.VMEM((6, 128), jnp.float32),
            pltpu.VMEM((6, 128), jnp.int32),
            pltpu.VMEM((_NUM_LEVELS, 128, 2 * _BLK), jnp.float32),
            pltpu.VMEM((_G, 2 * _NUM_LEVELS), jnp.float32),
            pltpu.SemaphoreType.DMA,
        ],
    )(_encode_body)


def kernel(positions, embeddings, aabb_min, aabb_max):
    aabb_min = aabb_min.astype(jnp.float32)
    inv = (1.0 / (aabb_max - aabb_min)).astype(jnp.float32)
    params = jnp.broadcast_to(jnp.concatenate([aabb_min, inv])[:, None], (6, 16))
    one = lax.optimization_barrier(jnp.float32(1.0))
    emb_blocks = embeddings.reshape(_TOTAL // _BLK, 2 * _BLK) * one
    return _build()(positions, emb_blocks, params) * one


# speedup vs baseline: 1.0135x; 1.0135x over previous
import functools

import jax
import jax.numpy as jnp
import numpy as np
from jax import lax
from jax.experimental import pallas as pl
from jax.experimental.pallas import tpu as pltpu
from jax.experimental.pallas import tpu_sc as plsc

_NUM_LEVELS = 16
_HASHMAP_SIZE = 2 ** 19
_N = 524288
_PRIME_X, _PRIME_Y, _PRIME_Z = 73856093, 19349663, 83492791

_RES, _OFF, _SIZE = [], [], []
_t = 0
for _l in range(_NUM_LEVELS):
    _r = min(int(16 * (2.0 ** _l)), 512)
    _RES.append(_r)
    _OFF.append(_t)
    _SIZE.append(min(_r ** 3, _HASHMAP_SIZE))
    _t += _SIZE[-1]
_TOTAL = _t

_NC, _NS = 2, 16
_NW = _NC * _NS
_G = 16
_PPW = _N // _NW
_NGROUPS = _PPW // _G
_BLK = 4

_GEOM_LEVELS = [[0], [1], [2], [3], [4], list(range(5, _NUM_LEVELS))]


def _encode_body(positions, emb_blocks, params, out,
                 pos_v, par_v, idx_v, w_v, col_v, rows_v, out_v, sem):
    wid = lax.axis_index("s") * np.int32(_NC) + lax.axis_index("c")
    wbase = wid * np.int32(_PPW)

    pltpu.sync_copy(params, par_v)
    iota = lax.iota(jnp.int32, 16)
    rowbase = iota * np.int32(2 * _NUM_LEVELS)
    amin = [par_v[i, :] for i in range(3)]
    ainv = [par_v[3 + i, :] for i in range(3)]
    rowids = [iota + np.int32(c * 16) for c in range(8)]
    one_i = jnp.full((16,), 1, jnp.int32)

    def group(g, base):
        base = pl.multiple_of(base, _G)
        pltpu.sync_copy(positions.at[pl.ds(base, _G)], pos_v)

        u = []
        for ax in range(3):
            p = plsc.load_gather(pos_v, [iota, jnp.full((16,), ax, jnp.int32)])
            u.append(jnp.clip((p - amin[ax]) * ainv[ax], 0.0, 1.0))

        for gi, levels in enumerate(_GEOM_LEVELS):
            res = _RES[levels[0]]
            mask = _SIZE[levels[0]] - 1
            s = [u[ax] * jnp.float32(res - 1) for ax in range(3)]
            c0 = [sv.astype(jnp.int32) for sv in s]
            f = [s[ax] - c0[ax].astype(jnp.float32) for ax in range(3)]
            c1 = [jnp.minimum(c0[ax] + np.int32(1), np.int32(res - 1))
                  for ax in range(3)]
            mult = (_PRIME_X, _PRIME_Y, _PRIME_Z)
            h0 = [c0[ax] * np.int32(mult[ax]) for ax in range(3)]
            h1 = [c1[ax] * np.int32(mult[ax]) for ax in range(3)]
            w0 = [1.0 - f[ax] for ax in range(3)]
            for dx in range(2):
                hx = h1[0] if dx else h0[0]
                wx = f[0] if dx else w0[0]
                for dy in range(2):
                    hxy = hx + (h1[1] if dy else h0[1])
                    wxy = wx * (f[1] if dy else w0[1])
                    for dz in range(2):
                        cidx = dx * 4 + dy * 2 + dz
                        hm = (hxy + (h1[2] if dz else h0[2])) & np.int32(mask)
                        w = wxy * (f[2] if dz else w0[2])
                        blk = lax.shift_right_logical(hm, np.int32(2))
                        col = lax.shift_left(hm & np.int32(_BLK - 1), np.int32(1))
                        for lvl in levels:
                            idx_v[lvl, cidx * 16:(cidx + 1) * 16] = (
                                blk + np.int32(_OFF[lvl] // _BLK))
                        w_v[gi, cidx * 16:(cidx + 1) * 16] = w
                        col_v[gi, cidx * 16:(cidx + 1) * 16] = col

        copies = [pltpu.async_copy(emb_blocks.at[idx_v.at[np.int32(l)]],
                                   rows_v.at[np.int32(l)], sem)
                  for l in range(_NUM_LEVELS)]
        for cp in copies:
            cp.wait()

        for gi, levels in enumerate(_GEOM_LEVELS):
            acc0 = [jnp.zeros((16,), jnp.float32) for _ in levels]
            acc1 = [jnp.zeros((16,), jnp.float32) for _ in levels]
            for cidx in range(8):
                w = w_v[gi, cidx * 16:(cidx + 1) * 16]
                col0 = col_v[gi, cidx * 16:(cidx + 1) * 16]
                col1 = col0 + one_i
                for li, lvl in enumerate(levels):
                    lsp = jnp.full((16,), lvl, jnp.int32)
                    f0 = plsc.load_gather(rows_v, [lsp, rowids[cidx], col0])
                    f1 = plsc.load_gather(rows_v, [lsp, rowids[cidx], col1])
                    acc0[li] = acc0[li] + w * f0
                    acc1[li] = acc1[li] + w * f1
            for li, lvl in enumerate(levels):
                plsc.store_scatter(out_v, [rowbase + np.int32(2 * lvl)], acc0[li])
                plsc.store_scatter(out_v, [rowbase + np.int32(2 * lvl + 1)], acc1[li])

        pltpu.sync_copy(out_v, out.at[pl.ds(base * np.int32(2 * _NUM_LEVELS),
                                            _G * 2 * _NUM_LEVELS)])
        return base + np.int32(_G)

    lax.fori_loop(0, _NGROUPS, group, wbase)


@functools.lru_cache(maxsize=1)
def _build():
    mesh = plsc.VectorSubcoreMesh(core_axis_name="c", subcore_axis_name="s")
    return functools.partial(
        pl.kernel,
        out_type=jax.ShapeDtypeStruct((_N * 2 * _NUM_LEVELS,), jnp.float32),
        mesh=mesh,
        compiler_params=pltpu.CompilerParams(needs_layout_passes=False,
                                             use_tc_tiling_on_sc=False),
        scratch_types=[
            pltpu.VMEM((_G, 3), jnp.float32),
            pltpu.VMEM((6, 16), jnp.float32),
            pltpu.VMEM((_NUM_LEVELS, 128), jnp.int32),
            pltpu.VMEM((6, 128), jnp.float32),
            pltpu.VMEM((6, 128), jnp.int32),
            pltpu.VMEM((_NUM_LEVELS, 128, 2 * _BLK), jnp.float32),
            pltpu.VMEM((_G * 2 * _NUM_LEVELS,), jnp.float32),
            pltpu.SemaphoreType.DMA,
        ],
    )(_encode_body)


def kernel(positions, embeddings, aabb_min, aabb_max):
    aabb_min = aabb_min.astype(jnp.float32)
    inv = (1.0 / (aabb_max - aabb_min)).astype(jnp.float32)
    params = jnp.broadcast_to(jnp.concatenate([aabb_min, inv])[:, None], (6, 16))
    emb_blocks = embeddings.reshape(_TOTAL // _BLK, 2 * _BLK)
    out_flat = _build()(positions, emb_blocks, params)
    return out_flat.reshape(_N, 2 * _NUM_LEVELS)


# speedup vs baseline: 1.8181x; 1.7939x over previous
import functools

import jax
import jax.numpy as jnp
import numpy as np
from jax import lax
from jax.experimental import pallas as pl
from jax.experimental.pallas import tpu as pltpu
from jax.experimental.pallas import tpu_sc as plsc

_NUM_LEVELS = 16
_HASHMAP_SIZE = 2 ** 19
_N = 524288
_PRIME_X, _PRIME_Y, _PRIME_Z = 73856093, 19349663, 83492791

_RES, _OFF, _SIZE = [], [], []
_t = 0
for _l in range(_NUM_LEVELS):
    _r = min(int(16 * (2.0 ** _l)), 512)
    _RES.append(_r)
    _OFF.append(_t)
    _SIZE.append(min(_r ** 3, _HASHMAP_SIZE))
    _t += _SIZE[-1]
_TOTAL = _t

_NC, _NS = 2, 16
_NW = _NC * _NS
_G = 16
_PPW = _N // _NW
_NGROUPS = _PPW // _G
_BLK = 4

_GEOM_LEVELS = [[0], [1], [2], [3], [4], list(range(5, _NUM_LEVELS))]


def _encode_body(positions, emb_blocks, params, out,
                 pos_v, par_v, idx_v, w_v, col_v, rows_v, out_v, sem):
    wid = lax.axis_index("s") * np.int32(_NC) + lax.axis_index("c")
    wbase = wid * np.int32(_PPW)

    pltpu.sync_copy(params, par_v)
    iota = lax.iota(jnp.int32, 16)
    rowbase = iota * np.int32(2 * _NUM_LEVELS)
    amin = [par_v[i, :] for i in range(3)]
    ainv = [par_v[3 + i, :] for i in range(3)]
    rowids = [iota + np.int32(c * 16) for c in range(8)]
    one_i = jnp.full((16,), 1, jnp.int32)

    def group(g, base):
        base = pl.multiple_of(base, _G)
        pltpu.sync_copy(positions.at[pl.ds(base, _G)], pos_v)

        u = []
        for ax in range(3):
            p = plsc.load_gather(pos_v, [iota, jnp.full((16,), ax, jnp.int32)])
            u.append(jnp.clip((p - amin[ax]) * ainv[ax], 0.0, 1.0))

        for gi, levels in enumerate(_GEOM_LEVELS):
            res = _RES[levels[0]]
            mask = _SIZE[levels[0]] - 1
            s = [u[ax] * jnp.float32(res - 1) for ax in range(3)]
            c0 = [sv.astype(jnp.int32) for sv in s]
            f = [s[ax] - c0[ax].astype(jnp.float32) for ax in range(3)]
            c1 = [jnp.minimum(c0[ax] + np.int32(1), np.int32(res - 1))
                  for ax in range(3)]
            mult = (_PRIME_X, _PRIME_Y, _PRIME_Z)
            h0 = [c0[ax] * np.int32(mult[ax]) for ax in range(3)]
            h1 = [c1[ax] * np.int32(mult[ax]) for ax in range(3)]
            w0 = [1.0 - f[ax] for ax in range(3)]
            for dx in range(2):
                hx = h1[0] if dx else h0[0]
                wx = f[0] if dx else w0[0]
                for dy in range(2):
                    hxy = hx + (h1[1] if dy else h0[1])
                    wxy = wx * (f[1] if dy else w0[1])
                    for dz in range(2):
                        cidx = dx * 4 + dy * 2 + dz
                        hm = (hxy + (h1[2] if dz else h0[2])) & np.int32(mask)
                        w = wxy * (f[2] if dz else w0[2])
                        f0base = (lax.shift_left(lax.shift_right_logical(hm, np.int32(7)),
                                                 np.int32(5))
                                  + (lax.shift_right_logical(hm, np.int32(3))
                                     & np.int32(15)))
                        col = hm & np.int32(7)
                        for lvl in levels:
                            r0 = f0base + np.int32(_OFF[lvl] // 4)
                            idx_v[2 * lvl, cidx * 16:(cidx + 1) * 16] = r0
                            idx_v[2 * lvl + 1, cidx * 16:(cidx + 1) * 16] = (
                                r0 + np.int32(16))
                        w_v[gi, cidx * 16:(cidx + 1) * 16] = w
                        col_v[gi, cidx * 16:(cidx + 1) * 16] = col

        copies = [pltpu.async_copy(emb_blocks.at[idx_v.at[np.int32(j)]],
                                   rows_v.at[np.int32(j)], sem)
                  for j in range(2 * _NUM_LEVELS)]
        for cp in copies:
            cp.wait()

        for gi, levels in enumerate(_GEOM_LEVELS):
            acc0 = [jnp.zeros((16,), jnp.float32) for _ in levels]
            acc1 = [jnp.zeros((16,), jnp.float32) for _ in levels]
            for cidx in range(8):
                w = w_v[gi, cidx * 16:(cidx + 1) * 16]
                col0 = col_v[gi, cidx * 16:(cidx + 1) * 16]
                for li, lvl in enumerate(levels):
                    f0 = plsc.load_gather(
                        rows_v, [jnp.full((16,), 2 * lvl, jnp.int32),
                                 rowids[cidx], col0])
                    f1 = plsc.load_gather(
                        rows_v, [jnp.full((16,), 2 * lvl + 1, jnp.int32),
                                 rowids[cidx], col0])
                    acc0[li] = acc0[li] + w * f0
                    acc1[li] = acc1[li] + w * f1
            for li, lvl in enumerate(levels):
                plsc.store_scatter(out_v, [rowbase + np.int32(2 * lvl)], acc0[li])
                plsc.store_scatter(out_v, [rowbase + np.int32(2 * lvl + 1)], acc1[li])

        pltpu.sync_copy(out_v, out.at[pl.ds(base * np.int32(2 * _NUM_LEVELS),
                                            _G * 2 * _NUM_LEVELS)])
        return base + np.int32(_G)

    lax.fori_loop(0, _NGROUPS, group, wbase)


@functools.lru_cache(maxsize=1)
def _build():
    mesh = plsc.VectorSubcoreMesh(core_axis_name="c", subcore_axis_name="s")
    return functools.partial(
        pl.kernel,
        out_type=jax.ShapeDtypeStruct((_N * 2 * _NUM_LEVELS,), jnp.float32),
        mesh=mesh,
        compiler_params=pltpu.CompilerParams(needs_layout_passes=False,
                                             use_tc_tiling_on_sc=False),
        scratch_types=[
            pltpu.VMEM((_G, 3), jnp.float32),
            pltpu.VMEM((6, 16), jnp.float32),
            pltpu.VMEM((2 * _NUM_LEVELS, 128), jnp.int32),
            pltpu.VMEM((6, 128), jnp.float32),
            pltpu.VMEM((6, 128), jnp.int32),
            pltpu.VMEM((2 * _NUM_LEVELS, 128, 8), jnp.float32),
            pltpu.VMEM((_G * 2 * _NUM_LEVELS,), jnp.float32),
            pltpu.SemaphoreType.DMA,
        ],
    )(_encode_body)


def kernel(positions, embeddings, aabb_min, aabb_max):
    aabb_min = aabb_min.astype(jnp.float32)
    inv = (1.0 / (aabb_max - aabb_min)).astype(jnp.float32)
    params = jnp.broadcast_to(jnp.concatenate([aabb_min, inv])[:, None], (6, 16))
    emb_pl = jnp.swapaxes(embeddings.reshape(_TOTAL // 128, 128, 2), 1, 2)
    emb_pl = emb_pl.reshape(_TOTAL * 2 // 8, 8)
    out_flat = _build()(positions, emb_pl, params)
    return out_flat.reshape(_N, 2 * _NUM_LEVELS)


# speedup vs baseline: 2.4058x; 1.3233x over previous
import functools

import jax
import jax.numpy as jnp
import numpy as np
from jax import lax
from jax.experimental import pallas as pl
from jax.experimental.pallas import tpu as pltpu
from jax.experimental.pallas import tpu_sc as plsc

_NUM_LEVELS = 16
_HASHMAP_SIZE = 2 ** 19
_N = 524288
_PRIME_X, _PRIME_Y, _PRIME_Z = 73856093, 19349663, 83492791

_RES, _OFF, _SIZE = [], [], []
_t = 0
for _l in range(_NUM_LEVELS):
    _r = min(int(16 * (2.0 ** _l)), 512)
    _RES.append(_r)
    _OFF.append(_t)
    _SIZE.append(min(_r ** 3, _HASHMAP_SIZE))
    _t += _SIZE[-1]
_TOTAL = _t

_NC, _NS = 2, 16
_NW = _NC * _NS
_G = 16
_PPW = _N // _NW
_NGROUPS = _PPW // _G
_ND = 2 * _NUM_LEVELS
_OW = 2 * _NUM_LEVELS

_GEOM_LEVELS = [[0], [1], [2], [3], [4], list(range(5, _NUM_LEVELS))]


def _encode_body(positions, emb_pl, params, out,
                 pos_v, par_v, idxA, idxB, wA, wB, colA, colB,
                 rowsA, rowsB, out_v, semA, semB):
    wid = lax.axis_index("s") * np.int32(_NC) + lax.axis_index("c")
    wbase = wid * np.int32(_PPW)
    wend = wbase + np.int32(_PPW)

    pltpu.sync_copy(params, par_v)
    iota = lax.iota(jnp.int32, 16)
    rowbase = iota * np.int32(_OW)
    amin = [par_v[i, :] for i in range(3)]
    ainv = [par_v[3 + i, :] for i in range(3)]
    rowids = [iota + np.int32(c * 16) for c in range(8)]

    def compute_fire(base, idxr, wr, colr, rowsr, sem):
        pltpu.sync_copy(positions.at[pl.ds(base, _G)], pos_v)
        u = []
        for ax in range(3):
            p = plsc.load_gather(pos_v, [iota, jnp.full((16,), ax, jnp.int32)])
            u.append(jnp.clip((p - amin[ax]) * ainv[ax], 0.0, 1.0))

        for gi, levels in enumerate(_GEOM_LEVELS):
            res = _RES[levels[0]]
            mask = _SIZE[levels[0]] - 1
            s = [u[ax] * jnp.float32(res - 1) for ax in range(3)]
            c0 = [sv.astype(jnp.int32) for sv in s]
            f = [s[ax] - c0[ax].astype(jnp.float32) for ax in range(3)]
            c1 = [jnp.minimum(c0[ax] + np.int32(1), np.int32(res - 1))
                  for ax in range(3)]
            mult = (_PRIME_X, _PRIME_Y, _PRIME_Z)
            h0 = [c0[ax] * np.int32(mult[ax]) for ax in range(3)]
            h1 = [c1[ax] * np.int32(mult[ax]) for ax in range(3)]
            w0 = [1.0 - f[ax] for ax in range(3)]
            for dx in range(2):
                hx = h1[0] if dx else h0[0]
                wx = f[0] if dx else w0[0]
                for dy in range(2):
                    hxy = hx + (h1[1] if dy else h0[1])
                    wxy = wx * (f[1] if dy else w0[1])
                    for dz in range(2):
                        cidx = dx * 4 + dy * 2 + dz
                        hm = (hxy + (h1[2] if dz else h0[2])) & np.int32(mask)
                        w = wxy * (f[2] if dz else w0[2])
                        f0base = (lax.shift_left(
                                      lax.shift_right_logical(hm, np.int32(7)),
                                      np.int32(5))
                                  + (lax.shift_right_logical(hm, np.int32(3))
                                     & np.int32(15)))
                        col = hm & np.int32(7)
                        for lvl in levels:
                            r0 = f0base + np.int32(_OFF[lvl] // 4)
                            idxr[2 * lvl, cidx * 16:(cidx + 1) * 16] = r0
                            idxr[2 * lvl + 1, cidx * 16:(cidx + 1) * 16] = (
                                r0 + np.int32(16))
                        wr[gi, cidx * 16:(cidx + 1) * 16] = w
                        colr[gi, cidx * 16:(cidx + 1) * 16] = col

        for j in range(_ND):
            pltpu.async_copy(emb_pl.at[idxr.at[np.int32(j)]],
                             rowsr.at[np.int32(j)], sem)

    def drain_acc(base, idxr, wr, colr, rowsr, sem):
        for j in range(_ND):
            pltpu.make_async_copy(emb_pl.at[idxr.at[np.int32(j)]],
                                  rowsr.at[np.int32(j)], sem).wait()
        for gi, levels in enumerate(_GEOM_LEVELS):
            acc0 = [jnp.zeros((16,), jnp.float32) for _ in levels]
            acc1 = [jnp.zeros((16,), jnp.float32) for _ in levels]
            for cidx in range(8):
                w = wr[gi, cidx * 16:(cidx + 1) * 16]
                col0 = colr[gi, cidx * 16:(cidx + 1) * 16]
                for li, lvl in enumerate(levels):
                    f0 = plsc.load_gather(
                        rowsr, [jnp.full((16,), 2 * lvl, jnp.int32),
                                rowids[cidx], col0])
                    f1 = plsc.load_gather(
                        rowsr, [jnp.full((16,), 2 * lvl + 1, jnp.int32),
                                rowids[cidx], col0])
                    acc0[li] = acc0[li] + w * f0
                    acc1[li] = acc1[li] + w * f1
            for li, lvl in enumerate(levels):
                plsc.store_scatter(out_v, [rowbase + np.int32(2 * lvl)], acc0[li])
                plsc.store_scatter(out_v, [rowbase + np.int32(2 * lvl + 1)], acc1[li])
        pltpu.sync_copy(out_v, out.at[pl.ds(base * np.int32(_OW), _G * _OW)])

    compute_fire(pl.multiple_of(wbase, _G), idxA, wA, colA, rowsA, semA)

    def pair(i, carry):
        baseA = pl.multiple_of(carry, _G)
        baseB = pl.multiple_of(carry + np.int32(_G), _G)
        baseA2 = pl.multiple_of(carry + np.int32(2 * _G), _G)
        compute_fire(baseB, idxB, wB, colB, rowsB, semB)
        drain_acc(baseA, idxA, wA, colA, rowsA, semA)
        @pl.when(baseA2 < wend)
        def _():
            compute_fire(baseA2, idxA, wA, colA, rowsA, semA)
        drain_acc(baseB, idxB, wB, colB, rowsB, semB)
        return carry + np.int32(2 * _G)

    lax.fori_loop(0, _NGROUPS // 2, pair, wbase)


@functools.lru_cache(maxsize=1)
def _build():
    mesh = plsc.VectorSubcoreMesh(core_axis_name="c", subcore_axis_name="s")
    return functools.partial(
        pl.kernel,
        out_type=jax.ShapeDtypeStruct((_N * _OW,), jnp.float32),
        mesh=mesh,
        compiler_params=pltpu.CompilerParams(needs_layout_passes=False,
                                             use_tc_tiling_on_sc=False),
        scratch_types=[
            pltpu.VMEM((_G, 3), jnp.float32),
            pltpu.VMEM((6, 16), jnp.float32),
            pltpu.VMEM((_ND, 128), jnp.int32),
            pltpu.VMEM((_ND, 128), jnp.int32),
            pltpu.VMEM((6, 128), jnp.float32),
            pltpu.VMEM((6, 128), jnp.float32),
            pltpu.VMEM((6, 128), jnp.int32),
            pltpu.VMEM((6, 128), jnp.int32),
            pltpu.VMEM((_ND, 128, 8), jnp.float32),
            pltpu.VMEM((_ND, 128, 8), jnp.float32),
            pltpu.VMEM((_G * _OW,), jnp.float32),
            pltpu.SemaphoreType.DMA,
            pltpu.SemaphoreType.DMA,
        ],
    )(_encode_body)


def kernel(positions, embeddings, aabb_min, aabb_max):
    aabb_min = aabb_min.astype(jnp.float32)
    inv = (1.0 / (aabb_max - aabb_min)).astype(jnp.float32)
    params = jnp.broadcast_to(jnp.concatenate([aabb_min, inv])[:, None], (6, 16))
    emb_pl = jnp.swapaxes(embeddings.reshape(_TOTAL // 128, 128, 2), 1, 2)
    emb_pl = emb_pl.reshape(_TOTAL * 2 // 8, 8)
    out_flat = _build()(positions, emb_pl, params)
    return out_flat.reshape(_N, _OW)


# speedup vs baseline: 3.6455x; 1.5153x over previous
import functools

import jax
import jax.numpy as jnp
import numpy as np
from jax import lax
from jax.experimental import pallas as pl
from jax.experimental.pallas import tpu as pltpu
from jax.experimental.pallas import tpu_sc as plsc

_NUM_LEVELS = 16
_HASHMAP_SIZE = 2 ** 19
_N = 524288
_PRIME_X, _PRIME_Y, _PRIME_Z = 73856093, 19349663, 83492791

_RES, _OFF, _SIZE = [], [], []
_t = 0
for _l in range(_NUM_LEVELS):
    _r = min(int(16 * (2.0 ** _l)), 512)
    _RES.append(_r)
    _OFF.append(_t)
    _SIZE.append(min(_r ** 3, _HASHMAP_SIZE))
    _t += _SIZE[-1]
_TOTAL = _t

_NC, _NS = 2, 16
_NW = _NC * _NS
_G = 16
_PPW = _N // _NW
_NGROUPS = _PPW // _G
_ND = _NUM_LEVELS
_OW = 2 * _NUM_LEVELS
_BLK = 4
_NBLOCKS = _TOTAL // 128
_BPT = _NBLOCKS // _NS
_CH = 9
assert _BPT % (2 * _CH) == 0

_GEOM_LEVELS = [[0], [1], [2], [3], [4], list(range(5, _NUM_LEVELS))]


def _encode_body(positions, emb_pl, params, out,
                 pos_v, par_v, idxA, idxB, wA, wB, colA, colB,
                 rowsA, rowsB, out_v, il_in, il_out, tbl, semA, semB, semI):
    sid = lax.axis_index("s")
    wid = sid * np.int32(_NC) + lax.axis_index("c")
    wbase = wid * np.int32(_PPW)
    wend = wbase + np.int32(_PPW)

    iota = lax.iota(jnp.int32, 16)

    il_inA, il_inB = il_in
    il_outA, il_outB = il_out
    semInA, semInB, semOutA, semOutB = semI
    il_row = lax.shift_right_logical(iota, np.int32(2))
    il_col0 = lax.shift_left(iota & np.int32(3), np.int32(1))
    il_col1 = il_col0 + np.int32(1)
    cstart = sid.astype(jnp.int32) * np.int32(_BPT)
    cend = cstart + np.int32(_BPT)

    def il_fire_in(cb, buf, sem):
        pltpu.async_copy(emb_pl.at[pl.ds(cb * np.int32(16), 16 * _CH)], buf, sem)

    def il_chunk(cb, inb, outb, sem_in, sem_out, first_cb_off):
        pltpu.make_async_copy(emb_pl.at[pl.ds(cb * np.int32(16), 16 * _CH)],
                              inb, sem_in).wait()
        @pl.when(cb >= cstart + np.int32(first_cb_off))
        def _():
            pltpu.make_async_copy(outb, tbl.at[pl.ds(cb * np.int32(32), 32 * _CH)],
                                  sem_out).wait()
        for blk in range(_CH):
            for k in range(8):
                v0 = inb[blk * 16 + k, :]
                v1 = inb[blk * 16 + 8 + k, :]
                rk = il_row + np.int32(blk * 32 + 4 * k)
                plsc.store_scatter(outb, [rk, il_col0], v0)
                plsc.store_scatter(outb, [rk, il_col1], v1)
        pltpu.async_copy(outb, tbl.at[pl.ds(cb * np.int32(32), 32 * _CH)], sem_out)

    il_fire_in(cstart, il_inA, semInA)

    def il_pair(i, cb):
        cbB = cb + np.int32(_CH)
        cbA2 = cb + np.int32(2 * _CH)
        il_fire_in(cbB, il_inB, semInB)
        il_chunk(cb, il_inA, il_outA, semInA, semOutA, 2 * _CH)

        @pl.when(cbA2 < cend)
        def _():
            il_fire_in(cbA2, il_inA, semInA)
        il_chunk(cbB, il_inB, il_outB, semInB, semOutB, 3 * _CH)
        return cbA2

    lax.fori_loop(0, _BPT // (2 * _CH), il_pair, cstart)
    pltpu.make_async_copy(il_outA, tbl.at[pl.ds(cstart * np.int32(32), 32 * _CH)],
                          semOutA).wait()
    pltpu.make_async_copy(il_outB, tbl.at[pl.ds(cstart * np.int32(32), 32 * _CH)],
                          semOutB).wait()
    plsc.subcore_barrier()

    pltpu.sync_copy(params, par_v)
    rowbase = iota * np.int32(_OW)
    amin = [par_v[i, :] for i in range(3)]
    ainv = [par_v[3 + i, :] for i in range(3)]
    rowids = [iota + np.int32(c * 16) for c in range(8)]
    one_i = jnp.full((16,), 1, jnp.int32)

    def compute_fire(base, idxr, wr, colr, rowsr, sem):
        pltpu.sync_copy(positions.at[pl.ds(base, _G)], pos_v)
        u = []
        for ax in range(3):
            p = plsc.load_gather(pos_v, [iota, jnp.full((16,), ax, jnp.int32)])
            u.append(jnp.clip((p - amin[ax]) * ainv[ax], 0.0, 1.0))

        for gi, levels in enumerate(_GEOM_LEVELS):
            res = _RES[levels[0]]
            mask = _SIZE[levels[0]] - 1
            s = [u[ax] * jnp.float32(res - 1) for ax in range(3)]
            c0 = [sv.astype(jnp.int32) for sv in s]
            f = [s[ax] - c0[ax].astype(jnp.float32) for ax in range(3)]
            c1 = [jnp.minimum(c0[ax] + np.int32(1), np.int32(res - 1))
                  for ax in range(3)]
            mult = (_PRIME_X, _PRIME_Y, _PRIME_Z)
            h0 = [c0[ax] * np.int32(mult[ax]) for ax in range(3)]
            h1 = [c1[ax] * np.int32(mult[ax]) for ax in range(3)]
            w0 = [1.0 - f[ax] for ax in range(3)]
            for dx in range(2):
                hx = h1[0] if dx else h0[0]
                wx = f[0] if dx else w0[0]
                for dy in range(2):
                    hxy = hx + (h1[1] if dy else h0[1])
                    wxy = wx * (f[1] if dy else w0[1])
                    for dz in range(2):
                        cidx = dx * 4 + dy * 2 + dz
                        hm = (hxy + (h1[2] if dz else h0[2])) & np.int32(mask)
                        w = wxy * (f[2] if dz else w0[2])
                        blk = lax.shift_right_logical(hm, np.int32(2))
                        col = lax.shift_left(hm & np.int32(_BLK - 1), np.int32(1))
                        for lvl in levels:
                            idxr[lvl, cidx * 16:(cidx + 1) * 16] = (
                                blk + np.int32(_OFF[lvl] // _BLK))
                        wr[gi, cidx * 16:(cidx + 1) * 16] = w
                        colr[gi, cidx * 16:(cidx + 1) * 16] = col

        for j in range(_ND):
            pltpu.async_copy(tbl.at[idxr.at[np.int32(j)]],
                             rowsr.at[np.int32(j)], sem)

    def drain_acc(base, idxr, wr, colr, rowsr, sem):
        for j in range(_ND):
            pltpu.make_async_copy(tbl.at[idxr.at[np.int32(j)]],
                                  rowsr.at[np.int32(j)], sem).wait()
        for gi, levels in enumerate(_GEOM_LEVELS):
            acc0 = [jnp.zeros((16,), jnp.float32) for _ in levels]
            acc1 = [jnp.zeros((16,), jnp.float32) for _ in levels]
            for cidx in range(8):
                w = wr[gi, cidx * 16:(cidx + 1) * 16]
                col0 = colr[gi, cidx * 16:(cidx + 1) * 16]
                col1 = col0 + one_i
                for li, lvl in enumerate(levels):
                    lsp = jnp.full((16,), lvl, jnp.int32)
                    f0 = plsc.load_gather(rowsr, [lsp, rowids[cidx], col0])
                    f1 = plsc.load_gather(rowsr, [lsp, rowids[cidx], col1])
                    acc0[li] = acc0[li] + w * f0
                    acc1[li] = acc1[li] + w * f1
            for li, lvl in enumerate(levels):
                plsc.store_scatter(out_v, [rowbase + np.int32(2 * lvl)], acc0[li])
                plsc.store_scatter(out_v, [rowbase + np.int32(2 * lvl + 1)], acc1[li])
        pltpu.sync_copy(out_v, out.at[pl.ds(base * np.int32(_OW), _G * _OW)])

    compute_fire(pl.multiple_of(wbase, _G), idxA, wA, colA, rowsA, semA)

    def pair(i, carry):
        baseA = pl.multiple_of(carry, _G)
        baseB = pl.multiple_of(carry + np.int32(_G), _G)
        baseA2 = pl.multiple_of(carry + np.int32(2 * _G), _G)
        compute_fire(baseB, idxB, wB, colB, rowsB, semB)
        drain_acc(baseA, idxA, wA, colA, rowsA, semA)
        @pl.when(baseA2 < wend)
        def _():
            compute_fire(baseA2, idxA, wA, colA, rowsA, semA)
        drain_acc(baseB, idxB, wB, colB, rowsB, semB)
        return carry + np.int32(2 * _G)

    lax.fori_loop(0, _NGROUPS // 2, pair, wbase)


@functools.lru_cache(maxsize=1)
def _build():
    mesh = plsc.VectorSubcoreMesh(core_axis_name="c", subcore_axis_name="s")
    return functools.partial(
        pl.kernel,
        out_type=jax.ShapeDtypeStruct((_N * _OW,), jnp.float32),
        mesh=mesh,
        compiler_params=pltpu.CompilerParams(needs_layout_passes=False,
                                             use_tc_tiling_on_sc=False),
        scratch_types=[
            pltpu.VMEM((_G, 3), jnp.float32),
            pltpu.VMEM((6, 16), jnp.float32),
            pltpu.VMEM((_ND, 128), jnp.int32),
            pltpu.VMEM((_ND, 128), jnp.int32),
            pltpu.VMEM((6, 128), jnp.float32),
            pltpu.VMEM((6, 128), jnp.float32),
            pltpu.VMEM((6, 128), jnp.int32),
            pltpu.VMEM((6, 128), jnp.int32),
            pltpu.VMEM((_ND, 128, 2 * _BLK), jnp.float32),
            pltpu.VMEM((_ND, 128, 2 * _BLK), jnp.float32),
            pltpu.VMEM((_G * _OW,), jnp.float32),
            (pltpu.VMEM((16 * _CH, 16), jnp.float32),
             pltpu.VMEM((16 * _CH, 16), jnp.float32)),
            (pltpu.VMEM((32 * _CH, 8), jnp.float32),
             pltpu.VMEM((32 * _CH, 8), jnp.float32)),
            pltpu.HBM((_TOTAL // _BLK, 2 * _BLK), jnp.float32),
            pltpu.SemaphoreType.DMA,
            pltpu.SemaphoreType.DMA,
            (pltpu.SemaphoreType.DMA, pltpu.SemaphoreType.DMA,
             pltpu.SemaphoreType.DMA, pltpu.SemaphoreType.DMA),
        ],
    )(_encode_body)


def kernel(positions, embeddings, aabb_min, aabb_max):
    aabb_min = aabb_min.astype(jnp.float32)
    inv = (1.0 / (aabb_max - aabb_min)).astype(jnp.float32)
    params = jnp.broadcast_to(jnp.concatenate([aabb_min, inv])[:, None], (6, 16))
    emb_pl = jnp.swapaxes(embeddings.reshape(_TOTAL // 128, 128, 2), 1, 2)
    emb_pl = emb_pl.reshape(_TOTAL * 2 // 16, 16)
    out_flat = _build()(positions, emb_pl, params)
    return out_flat.reshape(_N, _OW)
